# Initial kernel scaffold; baseline (speedup 1.0000x reference)
import jax
import jax.numpy as jnp
from jax.experimental import pallas as pl


def kernel(vertex_feat, edge_feat, edge_index, W_core_src, W_core_dst, W_core_bond, W_src_gate, W_dst_gate, W_bond_gate, g_core, b_core, g_gate, b_gate, W_out):
    raise NotImplementedError("write your pallas kernel here")


# speedup vs baseline: 3.1353x; 3.1353x over previous
import functools

import jax
import jax.numpy as jnp
from jax import lax
from jax.experimental import pallas as pl
from jax.experimental.pallas import tpu as pltpu
from jax.experimental.pallas import tpu_sc as plsc

N = 10000
E = 320000
D = 128
BD = 16
EPS = 1e-5

NC = 2
NS = 16
NW = NC * NS
EPW = E // NW


_BN = 2000


def _a_body(v_ref, w1_ref, w2_ref, w3_ref, w4_ref, s_ref, d_ref):
    v = v_ref[...]

    def mm(w_ref):
        return lax.dot_general(v, w_ref[...], (((1,), (1,)), ((), ())),
                               preferred_element_type=jnp.float32)

    s_ref[:, 0:D] = mm(w1_ref)
    s_ref[:, D:2 * D] = mm(w2_ref)
    d_ref[:, 0:D] = mm(w3_ref)
    d_ref[:, D:2 * D] = mm(w4_ref)


def _tables(v, w_cs, w_sg, w_cd, w_dg):
    wspec = pl.BlockSpec((D, D), lambda i: (0, 0))
    return pl.pallas_call(
        _a_body,
        grid=(N // _BN,),
        in_specs=[pl.BlockSpec((_BN, D), lambda i: (i, 0)),
                  wspec, wspec, wspec, wspec],
        out_specs=[pl.BlockSpec((_BN, 2 * D), lambda i: (i, 0)),
                   pl.BlockSpec((_BN, 2 * D), lambda i: (i, 0))],
        out_shape=[jax.ShapeDtypeStruct((N, 2 * D), jnp.float32),
                   jax.ShapeDtypeStruct((N, 2 * D), jnp.float32)],
    )(v, w_cs, w_sg, w_cd, w_dg)


_KB = 80
_NCH_B = EPW // _KB


def _b_body(stab_ref, dtab_ref, src_ref, dst_ref, out_ref,
            sidx, didx, srow, drow, sem1, sem2):
    wid = lax.axis_index("s") * NC + lax.axis_index("c")
    base = wid * EPW

    def chunk(i, _):
        off = base + i * _KB
        pltpu.sync_copy(src_ref.at[pl.ds(off, _KB)], sidx)
        pltpu.sync_copy(dst_ref.at[pl.ds(off, _KB)], didx)
        cp1 = pltpu.async_copy(stab_ref.at[sidx], srow, sem1)
        cp2 = pltpu.async_copy(dtab_ref.at[didx], drow, sem2)
        cp1.wait()
        cp2.wait()

        def addrow(r, _):
            for c in range(2 * D // 16):
                sl = pl.ds(c * 16, 16)
                srow[r, sl] = srow[r, sl] + drow[r, sl]
            return 0

        lax.fori_loop(0, _KB, addrow, 0)
        pltpu.sync_copy(srow, out_ref.at[pl.ds(off, _KB)])
        return 0

    lax.fori_loop(0, _NCH_B, chunk, 0)


def _gather_add(src_tab, dst_tab, src, dst):
    mesh = plsc.VectorSubcoreMesh(core_axis_name="c", subcore_axis_name="s",
                                  num_cores=NC, num_subcores=NS)
    fn = functools.partial(
        pl.kernel,
        out_type=jax.ShapeDtypeStruct((E, 2 * D), jnp.float32),
        mesh=mesh,
        scratch_types=[
            pltpu.VMEM((_KB,), jnp.int32),
            pltpu.VMEM((_KB,), jnp.int32),
            pltpu.VMEM((_KB, 2 * D), jnp.float32),
            pltpu.VMEM((_KB, 2 * D), jnp.float32),
            pltpu.SemaphoreType.DMA,
            pltpu.SemaphoreType.DMA,
        ],
    )(_b_body)
    return fn(src_tab, dst_tab, src, dst)


_BE = 2000


def _bonds(ef, wcb_ref, wbg_ref):
    bc = lax.dot_general(ef, wcb_ref[...], (((1,), (1,)), ((), ())),
                         preferred_element_type=jnp.float32)
    bg = lax.dot_general(ef, wbg_ref[...], (((1,), (1,)), ((), ())),
                         preferred_element_type=jnp.float32)
    return jnp.concatenate([bc, bg], axis=1)


def _c1_body(s_ref, ef_ref, wcb_ref, wbg_ref, out_ref):
    i = pl.program_id(0)

    @pl.when(i == 0)
    def _():
        out_ref[...] = jnp.zeros_like(out_ref)

    x = s_ref[...] + _bonds(ef_ref[...], wcb_ref, wbg_ref)
    out_ref[0:1, :] += jnp.sum(x, axis=0, keepdims=True)
    out_ref[1:2, :] += jnp.sum(x * x, axis=0, keepdims=True)


def _stats(s, ef, w_cb, w_bg):
    return pl.pallas_call(
        _c1_body,
        grid=(E // _BE,),
        in_specs=[pl.BlockSpec((_BE, 2 * D), lambda i: (i, 0)),
                  pl.BlockSpec((_BE, BD), lambda i: (i, 0)),
                  pl.BlockSpec((D, BD), lambda i: (0, 0)),
                  pl.BlockSpec((D, BD), lambda i: (0, 0))],
        out_specs=pl.BlockSpec((2, 2 * D), lambda i: (0, 0)),
        out_shape=jax.ShapeDtypeStruct((2, 2 * D), jnp.float32),
    )(s, ef, w_cb, w_bg)


def _c2_body(s_ref, ef_ref, wcb_ref, wbg_ref, st_ref, g_ref, b_ref, out_ref):
    x = s_ref[...] + _bonds(ef_ref[...], wcb_ref, wbg_ref)
    mean = st_ref[0:1, :] / E
    var = st_ref[1:2, :] / E - mean * mean
    a = g_ref[...] * lax.rsqrt(var + EPS)
    bb = b_ref[...] - mean * a
    xn = x * a + bb
    core = xn[:, 0:D]
    gate = xn[:, D:2 * D]
    sig_c = 1.0 / (1.0 + jnp.exp(-core))
    sig_g = 1.0 / (1.0 + jnp.exp(-gate))
    out_ref[...] = core * sig_c * sig_g


def _apply(s, ef, stats, w_cb, w_bg, gcat, bcat):
    return pl.pallas_call(
        _c2_body,
        grid=(E // _BE,),
        in_specs=[pl.BlockSpec((_BE, 2 * D), lambda i: (i, 0)),
                  pl.BlockSpec((_BE, BD), lambda i: (i, 0)),
                  pl.BlockSpec((D, BD), lambda i: (0, 0)),
                  pl.BlockSpec((D, BD), lambda i: (0, 0)),
                  pl.BlockSpec((2, 2 * D), lambda i: (0, 0)),
                  pl.BlockSpec((1, 2 * D), lambda i: (0, 0)),
                  pl.BlockSpec((1, 2 * D), lambda i: (0, 0))],
        out_specs=pl.BlockSpec((_BE, D), lambda i: (i, 0)),
        out_shape=jax.ShapeDtypeStruct((E, D), jnp.float32),
    )(s, ef, w_cb, w_bg, stats, gcat, bcat)


_KD = 80
_NCH_D = EPW // _KD
_N_PAD = 10240
_RPW = _N_PAD // NS
_ZR = 128


def _d_body(msg_ref, src_ref, out_ref, idx, mbuf, zbuf, acc):
    c = lax.axis_index("c")
    s = lax.axis_index("s")
    wid = s * NC + c
    base = wid * EPW

    def zrow(r, _):
        for k in range(D // 16):
            zbuf[r, pl.ds(k * 16, 16)] = jnp.zeros((16,), jnp.float32)
        return 0

    lax.fori_loop(0, _ZR, zrow, 0)
    for j in range(_RPW // _ZR):
        pltpu.sync_copy(zbuf, acc.at[pl.ds(s * _RPW + j * _ZR, _ZR)])
    plsc.subcore_barrier()

    def chunk(i, _):
        off = base + i * _KD
        pltpu.sync_copy(src_ref.at[pl.ds(off, _KD)], idx)
        pltpu.sync_copy(msg_ref.at[pl.ds(off, _KD)], mbuf)
        pltpu.sync_copy(mbuf, acc.at[idx], add=True)
        return 0

    lax.fori_loop(0, _NCH_D, chunk, 0)
    plsc.subcore_barrier()

    pltpu.sync_copy(acc.at[pl.ds(s * _RPW, _RPW)],
                    out_ref.at[c, pl.ds(s * _RPW, _RPW)])


def _scatter_add(msg, src):
    mesh = plsc.VectorSubcoreMesh(core_axis_name="c", subcore_axis_name="s",
                                  num_cores=NC, num_subcores=NS)
    fn = functools.partial(
        pl.kernel,
        out_type=jax.ShapeDtypeStruct((NC, _N_PAD, D), jnp.float32),
        mesh=mesh,
        scratch_types=[
            pltpu.VMEM((_KD,), jnp.int32),
            pltpu.VMEM((_KD, D), jnp.float32),
            pltpu.VMEM((_ZR, D), jnp.float32),
            pltpu.VMEM_SHARED((_N_PAD, D), jnp.float32),
        ],
    )(_d_body)
    return fn(msg, src)


def _e_body(p_ref, v_ref, w_ref, out_ref):
    accs = p_ref[0] + p_ref[1]
    out_ref[...] = lax.dot_general(
        accs, w_ref[...], (((1,), (1,)), ((), ())),
        preferred_element_type=jnp.float32) + v_ref[...]


def _finish(partials, w_out, v):
    return pl.pallas_call(
        _e_body,
        grid=(N // _BN,),
        in_specs=[pl.BlockSpec((NC, _BN, D), lambda i: (0, i, 0)),

                  pl.BlockSpec((_BN, D), lambda i: (i, 0)),
                  pl.BlockSpec((D, D), lambda i: (0, 0))],
        out_specs=pl.BlockSpec((_BN, D), lambda i: (i, 0)),
        out_shape=jax.ShapeDtypeStruct((N, D), jnp.float32),
    )(partials, v, w_out)


def kernel(vertex_feat, edge_feat, edge_index, W_core_src, W_core_dst,
           W_core_bond, W_src_gate, W_dst_gate, W_bond_gate, g_core, b_core,
           g_gate, b_gate, W_out):
    src = edge_index[0]
    dst = edge_index[1]
    src_tab, dst_tab = _tables(vertex_feat, W_core_src, W_src_gate,
                               W_core_dst, W_dst_gate)
    s = _gather_add(src_tab, dst_tab, src, dst)
    stats = _stats(s, edge_feat, W_core_bond, W_bond_gate)
    gcat = jnp.concatenate([g_core, g_gate]).reshape(1, 2 * D)
    bcat = jnp.concatenate([b_core, b_gate]).reshape(1, 2 * D)
    msg = _apply(s, edge_feat, stats, W_core_bond, W_bond_gate, gcat, bcat)
    partials = _scatter_add(msg, src)
    return _finish(partials, W_out, vertex_feat)


# speedup vs baseline: 3.7831x; 1.2066x over previous
import functools

import jax
import jax.numpy as jnp
from jax import lax
from jax.experimental import pallas as pl
from jax.experimental.pallas import tpu as pltpu
from jax.experimental.pallas import tpu_sc as plsc

N = 10000
E = 320000
D = 128
BD = 16
EPS = 1e-5

NC = 2
NS = 16
NW = NC * NS
EPW = E // NW

_HI_MASK = -65536


def _pack2(lo_f32, hi_f32):
    lo = lax.bitcast_convert_type(lo_f32.astype(jnp.bfloat16), jnp.uint16)
    hi = lax.bitcast_convert_type(hi_f32.astype(jnp.bfloat16), jnp.uint16)
    return (hi.astype(jnp.int32) << 16) | lo.astype(jnp.int32)


def _unpack_lo(w):
    return lax.bitcast_convert_type(w << 16, jnp.float32)


def _unpack_hi(w):
    return lax.bitcast_convert_type(w & _HI_MASK, jnp.float32)


_BN = 2000


def _a_body(v_ref, w1_ref, w2_ref, w3_ref, w4_ref, s_ref, d_ref):
    v = v_ref[...]

    def mm(w_ref):
        return lax.dot_general(v, w_ref[...], (((1,), (1,)), ((), ())),
                               preferred_element_type=jnp.float32)

    s_ref[...] = _pack2(mm(w1_ref), mm(w2_ref))
    d_ref[...] = _pack2(mm(w3_ref), mm(w4_ref))


def _tables(v, w_cs, w_sg, w_cd, w_dg):
    wspec = pl.BlockSpec((D, D), lambda i: (0, 0))
    return pl.pallas_call(
        _a_body,
        grid=(N // _BN,),
        in_specs=[pl.BlockSpec((_BN, D), lambda i: (i, 0)),
                  wspec, wspec, wspec, wspec],
        out_specs=[pl.BlockSpec((_BN, D), lambda i: (i, 0)),
                   pl.BlockSpec((_BN, D), lambda i: (i, 0))],
        out_shape=[jax.ShapeDtypeStruct((N, D), jnp.int32),
                   jax.ShapeDtypeStruct((N, D), jnp.int32)],
    )(v, w_cs, w_sg, w_cd, w_dg)


_KB = 80
_NCH_B = EPW // _KB


def _b_body(stab_ref, dtab_ref, src_ref, dst_ref, out_ref,
            sidx, didx, srow, drow, sem1, sem2):
    wid = lax.axis_index("s") * NC + lax.axis_index("c")
    base = wid * EPW

    def chunk(i, _):
        off = base + i * _KB
        pltpu.sync_copy(src_ref.at[pl.ds(off, _KB)], sidx)
        pltpu.sync_copy(dst_ref.at[pl.ds(off, _KB)], didx)
        cp1 = pltpu.async_copy(stab_ref.at[sidx], srow, sem1)
        cp2 = pltpu.async_copy(dtab_ref.at[didx], drow, sem2)
        cp1.wait()
        cp2.wait()

        def addrow(r, _):
            for c in range(D // 16):
                sl = pl.ds(c * 16, 16)
                aw = srow[r, sl]
                bw = drow[r, sl]
                bc = lax.bitcast_convert_type
                lo = (bc(aw << 16, jnp.float32) + bc(bw << 16, jnp.float32))
                hi = (bc(aw & _HI_MASK, jnp.float32)
                      + bc(bw & _HI_MASK, jnp.float32))
                srow[r, sl] = (
                    (bc(hi, jnp.int32) & _HI_MASK)
                    | lax.shift_right_logical(bc(lo, jnp.int32), 16))
            return 0

        lax.fori_loop(0, _KB, addrow, 0)
        pltpu.sync_copy(srow, out_ref.at[pl.ds(off, _KB)])
        return 0

    lax.fori_loop(0, _NCH_B, chunk, 0)


def _gather_add(src_tab, dst_tab, src, dst):
    mesh = plsc.VectorSubcoreMesh(core_axis_name="c", subcore_axis_name="s",
                                  num_cores=NC, num_subcores=NS)
    fn = functools.partial(
        pl.kernel,
        out_type=jax.ShapeDtypeStruct((E, D), jnp.int32),
        mesh=mesh,
        scratch_types=[
            pltpu.VMEM((_KB,), jnp.int32),
            pltpu.VMEM((_KB,), jnp.int32),
            pltpu.VMEM((_KB, D), jnp.int32),
            pltpu.VMEM((_KB, D), jnp.int32),
            pltpu.SemaphoreType.DMA,
            pltpu.SemaphoreType.DMA,
        ],
    )(_b_body)
    return fn(src_tab, dst_tab, src, dst)


_BE = 2000


def _bond(ef, w_ref):
    return lax.dot_general(ef, w_ref[...], (((1,), (1,)), ((), ())),
                           preferred_element_type=jnp.float32)


def _c1_body(s_ref, ef_ref, wcb_ref, wbg_ref, out_ref):
    i = pl.program_id(0)

    @pl.when(i == 0)
    def _():
        out_ref[...] = jnp.zeros_like(out_ref)

    w = s_ref[...]
    ef = ef_ref[...]
    core = _unpack_lo(w) + _bond(ef, wcb_ref)
    gate = _unpack_hi(w) + _bond(ef, wbg_ref)
    out_ref[0:1, :] += jnp.sum(core, axis=0, keepdims=True)
    out_ref[1:2, :] += jnp.sum(core * core, axis=0, keepdims=True)
    out_ref[2:3, :] += jnp.sum(gate, axis=0, keepdims=True)
    out_ref[3:4, :] += jnp.sum(gate * gate, axis=0, keepdims=True)


def _stats(s, ef, w_cb, w_bg):
    return pl.pallas_call(
        _c1_body,
        grid=(E // _BE,),
        in_specs=[pl.BlockSpec((_BE, D), lambda i: (i, 0)),
                  pl.BlockSpec((_BE, BD), lambda i: (i, 0)),
                  pl.BlockSpec((D, BD), lambda i: (0, 0)),
                  pl.BlockSpec((D, BD), lambda i: (0, 0))],
        out_specs=pl.BlockSpec((4, D), lambda i: (0, 0)),
        out_shape=jax.ShapeDtypeStruct((4, D), jnp.float32),
    )(s, ef, w_cb, w_bg)


def _c2_body(s_ref, ef_ref, wcb_ref, wbg_ref, st_ref, gc_ref, bc_ref,
             gg_ref, bg_ref, out_ref):
    w = s_ref[...]
    ef = ef_ref[...]
    core = _unpack_lo(w) + _bond(ef, wcb_ref)
    gate = _unpack_hi(w) + _bond(ef, wbg_ref)

    mean_c = st_ref[0:1, :] / E
    var_c = st_ref[1:2, :] / E - mean_c * mean_c
    a_c = gc_ref[...] * lax.rsqrt(var_c + EPS)
    b_c = bc_ref[...] - mean_c * a_c

    mean_g = st_ref[2:3, :] / E
    var_g = st_ref[3:4, :] / E - mean_g * mean_g
    a_g = gg_ref[...] * lax.rsqrt(var_g + EPS)
    b_g = bg_ref[...] - mean_g * a_g

    cn = core * a_c + b_c
    gn = gate * a_g + b_g
    sig_c = 1.0 / (1.0 + jnp.exp(-cn))
    sig_g = 1.0 / (1.0 + jnp.exp(-gn))
    out_ref[...] = cn * sig_c * sig_g


def _apply(s, ef, stats, w_cb, w_bg, g_core, b_core, g_gate, b_gate):
    pspec = pl.BlockSpec((1, D), lambda i: (0, 0))
    return pl.pallas_call(
        _c2_body,
        grid=(E // _BE,),
        in_specs=[pl.BlockSpec((_BE, D), lambda i: (i, 0)),
                  pl.BlockSpec((_BE, BD), lambda i: (i, 0)),
                  pl.BlockSpec((D, BD), lambda i: (0, 0)),
                  pl.BlockSpec((D, BD), lambda i: (0, 0)),
                  pl.BlockSpec((4, D), lambda i: (0, 0)),
                  pspec, pspec, pspec, pspec],
        out_specs=pl.BlockSpec((_BE, D), lambda i: (i, 0)),
        out_shape=jax.ShapeDtypeStruct((E, D), jnp.float32),
    )(s, ef, w_cb, w_bg, stats, g_core, b_core, g_gate, b_gate)


_KD = 80
_NCH_D = EPW // _KD
_N_PAD = 10240
_RPW = _N_PAD // NS
_ZR = 128


def _d_body(msg_ref, src_ref, out_ref, idx, mbuf, zbuf, acc):
    c = lax.axis_index("c")
    s = lax.axis_index("s")
    wid = s * NC + c
    base = wid * EPW

    def zrow(r, _):
        for k in range(D // 16):
            zbuf[r, pl.ds(k * 16, 16)] = jnp.zeros((16,), jnp.float32)
        return 0

    lax.fori_loop(0, _ZR, zrow, 0)
    for j in range(_RPW // _ZR):
        pltpu.sync_copy(zbuf, acc.at[pl.ds(s * _RPW + j * _ZR, _ZR)])
    plsc.subcore_barrier()

    def chunk(i, _):
        off = base + i * _KD
        pltpu.sync_copy(src_ref.at[pl.ds(off, _KD)], idx)
        pltpu.sync_copy(msg_ref.at[pl.ds(off, _KD)], mbuf)
        pltpu.sync_copy(mbuf, acc.at[idx], add=True)
        return 0

    lax.fori_loop(0, _NCH_D, chunk, 0)
    plsc.subcore_barrier()

    pltpu.sync_copy(acc.at[pl.ds(s * _RPW, _RPW)],
                    out_ref.at[c, pl.ds(s * _RPW, _RPW)])


def _scatter_add(msg, src):
    mesh = plsc.VectorSubcoreMesh(core_axis_name="c", subcore_axis_name="s",
                                  num_cores=NC, num_subcores=NS)
    fn = functools.partial(
        pl.kernel,
        out_type=jax.ShapeDtypeStruct((NC, _N_PAD, D), jnp.float32),
        mesh=mesh,
        scratch_types=[
            pltpu.VMEM((_KD,), jnp.int32),
            pltpu.VMEM((_KD, D), jnp.float32),
            pltpu.VMEM((_ZR, D), jnp.float32),
            pltpu.VMEM_SHARED((_N_PAD, D), jnp.float32),
        ],
    )(_d_body)
    return fn(msg, src)


def _e_body(p_ref, v_ref, w_ref, out_ref):
    accs = p_ref[0] + p_ref[1]
    out_ref[...] = lax.dot_general(
        accs, w_ref[...], (((1,), (1,)), ((), ())),
        preferred_element_type=jnp.float32) + v_ref[...]


def _finish(partials, w_out, v):
    return pl.pallas_call(
        _e_body,
        grid=(N // _BN,),
        in_specs=[pl.BlockSpec((NC, _BN, D), lambda i: (0, i, 0)),
                  pl.BlockSpec((_BN, D), lambda i: (i, 0)),
                  pl.BlockSpec((D, D), lambda i: (0, 0))],
        out_specs=pl.BlockSpec((_BN, D), lambda i: (i, 0)),
        out_shape=jax.ShapeDtypeStruct((N, D), jnp.float32),
    )(partials, v, w_out)


def kernel(vertex_feat, edge_feat, edge_index, W_core_src, W_core_dst,
           W_core_bond, W_src_gate, W_dst_gate, W_bond_gate, g_core, b_core,
           g_gate, b_gate, W_out):
    src = edge_index[0]
    dst = edge_index[1]
    src_tab, dst_tab = _tables(vertex_feat, W_core_src, W_src_gate,
                               W_core_dst, W_dst_gate)
    s = _gather_add(src_tab, dst_tab, src, dst)
    stats = _stats(s, edge_feat, W_core_bond, W_bond_gate)
    msg = _apply(s, edge_feat, stats, W_core_bond, W_bond_gate,
                 g_core.reshape(1, D), b_core.reshape(1, D),
                 g_gate.reshape(1, D), b_gate.reshape(1, D))
    partials = _scatter_add(msg, src)
    return _finish(partials, W_out, vertex_feat)


# speedup vs baseline: 5.6103x; 1.4830x over previous
import functools

import jax
import jax.numpy as jnp
from jax import lax
from jax.experimental import pallas as pl
from jax.experimental.pallas import tpu as pltpu
from jax.experimental.pallas import tpu_sc as plsc

N = 10000
E = 320000
D = 128
BD = 16
EPS = 1e-5

NC = 2
NS = 16
NW = NC * NS
EPW = E // NW

_HI_MASK = -65536


def _pack2(lo_f32, hi_f32):
    lo = lax.bitcast_convert_type(lo_f32.astype(jnp.bfloat16), jnp.uint16)
    hi = lax.bitcast_convert_type(hi_f32.astype(jnp.bfloat16), jnp.uint16)
    return (hi.astype(jnp.int32) << 16) | lo.astype(jnp.int32)


def _unpack_lo(w):
    return lax.bitcast_convert_type(w << 16, jnp.float32)


def _unpack_hi(w):
    return lax.bitcast_convert_type(w & _HI_MASK, jnp.float32)


_BN = 2000


def _a_body(v_ref, w1_ref, w2_ref, w3_ref, w4_ref, s_ref, d_ref):
    v = v_ref[...]

    def mm(w_ref):
        return lax.dot_general(v, w_ref[...], (((1,), (1,)), ((), ())),
                               preferred_element_type=jnp.float32)

    s_ref[...] = _pack2(mm(w1_ref), mm(w2_ref))
    d_ref[...] = _pack2(mm(w3_ref), mm(w4_ref))


def _tables(v, w_cs, w_sg, w_cd, w_dg):
    wspec = pl.BlockSpec((D, D), lambda i: (0, 0))
    return pl.pallas_call(
        _a_body,
        grid=(N // _BN,),
        in_specs=[pl.BlockSpec((_BN, D), lambda i: (i, 0)),
                  wspec, wspec, wspec, wspec],
        out_specs=[pl.BlockSpec((_BN, D), lambda i: (i, 0)),
                   pl.BlockSpec((_BN, D), lambda i: (i, 0))],
        out_shape=[jax.ShapeDtypeStruct((N, D), jnp.int32),
                   jax.ShapeDtypeStruct((N, D), jnp.int32)],
    )(v, w_cs, w_sg, w_cd, w_dg)


_KB = 80
_NCH_B = EPW // _KB


def _b_body(stab_ref, dtab_ref, src_ref, dst_ref, out_ref,
            sidx_all, didx_all,
            srow0, srow1, srow2, srow3, drow0, drow1, drow2, drow3,
            gsem0, gsem1, gsem2, gsem3, wsem0, wsem1, wsem2, wsem3):
    wid = lax.axis_index("s") * NC + lax.axis_index("c")
    base = wid * EPW
    srow = [srow0, srow1, srow2, srow3]
    drow = [drow0, drow1, drow2, drow3]
    gsem = [gsem0, gsem1, gsem2, gsem3]
    wsem = [wsem0, wsem1, wsem2, wsem3]

    pltpu.sync_copy(src_ref.at[pl.ds(base, EPW)], sidx_all)
    pltpu.sync_copy(dst_ref.at[pl.ds(base, EPW)], didx_all)

    def issue_gather(i, j):
        s_sl = sidx_all.at[pl.ds(i * _KB, _KB)]
        d_sl = didx_all.at[pl.ds(i * _KB, _KB)]
        pltpu.async_copy(stab_ref.at[s_sl], srow[j], gsem[j])
        pltpu.async_copy(dtab_ref.at[d_sl], drow[j], gsem[j])

    def wait_gather(j):
        s_sl = sidx_all.at[pl.ds(0, _KB)]
        pltpu.make_async_copy(stab_ref.at[s_sl], srow[j], gsem[j]).wait()
        pltpu.make_async_copy(dtab_ref.at[s_sl], drow[j], gsem[j]).wait()

    def add_rows(j):
        sj, dj = srow[j], drow[j]

        def addrow(r, _):
            for c in range(D // 16):
                sl = pl.ds(c * 16, 16)
                aw = sj[r, sl]
                bw = dj[r, sl]
                bc = lax.bitcast_convert_type
                lo = (bc(aw << 16, jnp.float32) + bc(bw << 16, jnp.float32))
                hi = (bc(aw & _HI_MASK, jnp.float32)
                      + bc(bw & _HI_MASK, jnp.float32))
                sj[r, sl] = (
                    (bc(hi, jnp.int32) & _HI_MASK)
                    | lax.shift_right_logical(bc(lo, jnp.int32), 16))
            return 0

        lax.fori_loop(0, _KB, addrow, 0)

    def issue_writeout(i, j):
        pltpu.async_copy(srow[j], out_ref.at[pl.ds(base + i * _KB, _KB)],
                         wsem[j])

    def wait_writeout(j):
        pltpu.make_async_copy(srow[j], out_ref.at[pl.ds(base, _KB)],
                              wsem[j]).wait()

    for c, j in ((0, 1), (1, 2), (2, 3)):
        issue_gather(c, j)
        wait_gather(j)
        add_rows(j)
        issue_writeout(c, j)
    issue_gather(3, 0)
    wait_writeout(1)
    issue_gather(4, 1)

    def step(q, _):
        for j in range(4):
            c = 3 + 4 * q + j
            wait_gather(j)
            add_rows(j)
            issue_writeout(c, j)
            jp = (j + 2) % 4
            wait_writeout(jp)
            issue_gather(c + 2, jp)
        return 0

    lax.fori_loop(0, (_NCH_B - 5) // 4, step, 0)

    for c, j in ((_NCH_B - 2, 0), (_NCH_B - 1, 1)):
        wait_gather(j)
        add_rows(j)
        issue_writeout(c, j)
    for j in (2, 3, 0, 1):
        wait_writeout(j)


def _gather_add(src_tab, dst_tab, src, dst):
    mesh = plsc.VectorSubcoreMesh(core_axis_name="c", subcore_axis_name="s",
                                  num_cores=NC, num_subcores=NS)
    fn = functools.partial(
        pl.kernel,
        out_type=jax.ShapeDtypeStruct((E, D), jnp.int32),
        mesh=mesh,
        scratch_types=(
            [pltpu.VMEM((EPW,), jnp.int32), pltpu.VMEM((EPW,), jnp.int32)]
            + [pltpu.VMEM((_KB, D), jnp.int32)] * 8
            + [pltpu.SemaphoreType.DMA] * 8
        ),
    )(_b_body)
    return fn(src_tab, dst_tab, src, dst)


_BE = 2000


def _bond(ef, w_ref):
    return lax.dot_general(ef, w_ref[...], (((1,), (1,)), ((), ())),
                           preferred_element_type=jnp.float32)


def _c1_body(s_ref, ef_ref, wcb_ref, wbg_ref, out_ref):
    i = pl.program_id(0)

    @pl.when(i == 0)
    def _():
        out_ref[...] = jnp.zeros_like(out_ref)

    w = s_ref[...]
    ef = ef_ref[...]
    core = _unpack_lo(w) + _bond(ef, wcb_ref)
    gate = _unpack_hi(w) + _bond(ef, wbg_ref)
    out_ref[0:1, :] += jnp.sum(core, axis=0, keepdims=True)
    out_ref[1:2, :] += jnp.sum(core * core, axis=0, keepdims=True)
    out_ref[2:3, :] += jnp.sum(gate, axis=0, keepdims=True)
    out_ref[3:4, :] += jnp.sum(gate * gate, axis=0, keepdims=True)


def _stats(s, ef, w_cb, w_bg):
    return pl.pallas_call(
        _c1_body,
        grid=(E // _BE,),
        in_specs=[pl.BlockSpec((_BE, D), lambda i: (i, 0)),
                  pl.BlockSpec((_BE, BD), lambda i: (i, 0)),
                  pl.BlockSpec((D, BD), lambda i: (0, 0)),
                  pl.BlockSpec((D, BD), lambda i: (0, 0))],
        out_specs=pl.BlockSpec((4, D), lambda i: (0, 0)),
        out_shape=jax.ShapeDtypeStruct((4, D), jnp.float32),
    )(s, ef, w_cb, w_bg)


def _c2_body(s_ref, ef_ref, wcb_ref, wbg_ref, st_ref, gc_ref, bc_ref,
             gg_ref, bg_ref, out_ref):
    w = s_ref[...]
    ef = ef_ref[...]
    core = _unpack_lo(w) + _bond(ef, wcb_ref)
    gate = _unpack_hi(w) + _bond(ef, wbg_ref)

    mean_c = st_ref[0:1, :] / E
    var_c = st_ref[1:2, :] / E - mean_c * mean_c
    a_c = gc_ref[...] * lax.rsqrt(var_c + EPS)
    b_c = bc_ref[...] - mean_c * a_c

    mean_g = st_ref[2:3, :] / E
    var_g = st_ref[3:4, :] / E - mean_g * mean_g
    a_g = gg_ref[...] * lax.rsqrt(var_g + EPS)
    b_g = bg_ref[...] - mean_g * a_g

    cn = core * a_c + b_c
    gn = gate * a_g + b_g
    sig_c = 1.0 / (1.0 + jnp.exp(-cn))
    sig_g = 1.0 / (1.0 + jnp.exp(-gn))
    out_ref[...] = cn * sig_c * sig_g


def _apply(s, ef, stats, w_cb, w_bg, g_core, b_core, g_gate, b_gate):
    pspec = pl.BlockSpec((1, D), lambda i: (0, 0))
    return pl.pallas_call(
        _c2_body,
        grid=(E // _BE,),
        in_specs=[pl.BlockSpec((_BE, D), lambda i: (i, 0)),
                  pl.BlockSpec((_BE, BD), lambda i: (i, 0)),
                  pl.BlockSpec((D, BD), lambda i: (0, 0)),
                  pl.BlockSpec((D, BD), lambda i: (0, 0)),
                  pl.BlockSpec((4, D), lambda i: (0, 0)),
                  pspec, pspec, pspec, pspec],
        out_specs=pl.BlockSpec((_BE, D), lambda i: (i, 0)),
        out_shape=jax.ShapeDtypeStruct((E, D), jnp.float32),
    )(s, ef, w_cb, w_bg, stats, g_core, b_core, g_gate, b_gate)


_KD = 80
_NCH_D = EPW // _KD
_N_PAD = 10240
_RPW = _N_PAD // NS
_ZR = 128


def _d_body(msg_ref, src_ref, out_ref, idx0, idx1, mbuf0, mbuf1, zbuf, acc,
            csem0, csem1):
    c = lax.axis_index("c")
    s = lax.axis_index("s")
    wid = s * NC + c
    base = wid * EPW
    idxv = [idx0, idx1]
    mbuf = [mbuf0, mbuf1]
    csem = [csem0, csem1]

    def zrow(r, _):
        for k in range(D // 16):
            zbuf[r, pl.ds(k * 16, 16)] = jnp.zeros((16,), jnp.float32)
        return 0

    lax.fori_loop(0, _ZR, zrow, 0)
    for j in range(_RPW // _ZR):
        pltpu.sync_copy(zbuf, acc.at[pl.ds(s * _RPW + j * _ZR, _ZR)])
    plsc.subcore_barrier()

    def issue_copies(i, j):
        off = base + i * _KD
        pltpu.async_copy(src_ref.at[pl.ds(off, _KD)], idxv[j], csem[j])
        pltpu.async_copy(msg_ref.at[pl.ds(off, _KD)], mbuf[j], csem[j])

    def wait_copies(j):
        pltpu.make_async_copy(src_ref.at[pl.ds(0, _KD)], idxv[j],
                              csem[j]).wait()
        pltpu.make_async_copy(msg_ref.at[pl.ds(0, _KD)], mbuf[j],
                              csem[j]).wait()

    def scatter(j):
        pltpu.sync_copy(mbuf[j], acc.at[idxv[j]], add=True)

    issue_copies(0, 0)
    issue_copies(1, 1)

    def pair(p, _):
        for j in range(2):
            i = 2 * p + j
            wait_copies(j)
            scatter(j)
            issue_copies(i + 2, j)
        return 0

    lax.fori_loop(0, (_NCH_D - 3) // 2, pair, 0)

    wait_copies(0)
    scatter(0)
    issue_copies(_NCH_D - 1, 0)
    wait_copies(1)
    scatter(1)
    wait_copies(0)
    scatter(0)

    plsc.subcore_barrier()

    pltpu.sync_copy(acc.at[pl.ds(s * _RPW, _RPW)],
                    out_ref.at[c, pl.ds(s * _RPW, _RPW)])


def _scatter_add(msg, src):
    mesh = plsc.VectorSubcoreMesh(core_axis_name="c", subcore_axis_name="s",
                                  num_cores=NC, num_subcores=NS)
    fn = functools.partial(
        pl.kernel,
        out_type=jax.ShapeDtypeStruct((NC, _N_PAD, D), jnp.float32),
        mesh=mesh,
        scratch_types=[
            pltpu.VMEM((_KD,), jnp.int32),
            pltpu.VMEM((_KD,), jnp.int32),
            pltpu.VMEM((_KD, D), jnp.float32),
            pltpu.VMEM((_KD, D), jnp.float32),
            pltpu.VMEM((_ZR, D), jnp.float32),
            pltpu.VMEM_SHARED((_N_PAD, D), jnp.float32),
            pltpu.SemaphoreType.DMA,
            pltpu.SemaphoreType.DMA,
        ],
    )(_d_body)
    return fn(msg, src)


def _e_body(p_ref, v_ref, w_ref, out_ref):
    accs = p_ref[0] + p_ref[1]
    out_ref[...] = lax.dot_general(
        accs, w_ref[...], (((1,), (1,)), ((), ())),
        preferred_element_type=jnp.float32) + v_ref[...]


def _finish(partials, w_out, v):
    return pl.pallas_call(
        _e_body,
        grid=(N // _BN,),
        in_specs=[pl.BlockSpec((NC, _BN, D), lambda i: (0, i, 0)),
                  pl.BlockSpec((_BN, D), lambda i: (i, 0)),
                  pl.BlockSpec((D, D), lambda i: (0, 0))],
        out_specs=pl.BlockSpec((_BN, D), lambda i: (i, 0)),
        out_shape=jax.ShapeDtypeStruct((N, D), jnp.float32),
    )(partials, v, w_out)


def kernel(vertex_feat, edge_feat, edge_index, W_core_src, W_core_dst,
           W_core_bond, W_src_gate, W_dst_gate, W_bond_gate, g_core, b_core,
           g_gate, b_gate, W_out):
    src = edge_index[0]
    dst = edge_index[1]
    src_tab, dst_tab = _tables(vertex_feat, W_core_src, W_src_gate,
                               W_core_dst, W_dst_gate)
    s = _gather_add(src_tab, dst_tab, src, dst)
    stats = _stats(s, edge_feat, W_core_bond, W_bond_gate)
    msg = _apply(s, edge_feat, stats, W_core_bond, W_bond_gate,
                 g_core.reshape(1, D), b_core.reshape(1, D),
                 g_gate.reshape(1, D), b_gate.reshape(1, D))
    partials = _scatter_add(msg, src)
    return _finish(partials, W_out, vertex_feat)
